# baseline (speedup 1.0000x reference)
import functools

import jax
import jax.numpy as jnp
from jax import lax
from jax.experimental import pallas as pl
from jax.experimental.pallas import tpu as pltpu
from jax.experimental.pallas import tpu_sc as plsc

BSZ, BEAM, V = 32, 8, 32768
K = 2 * BEAM
NROW = BSZ * BEAM
NW = 32
RPW = NROW // NW
NGRP = V // 256
NEG = float("-inf")
BIG = 1 << 30


def _sc_body(lp0_hbm, lp1_hbm, sc0_hbm, sc1_hbm,
             tv0_hbm, ti0_hbm, tv1_hbm, ti1_hbm,
             row_v, scv, tmpv, tmpi):
    wid = lax.axis_index("s") * 2 + lax.axis_index("c")
    base = wid * RPW
    lane = lax.iota(jnp.int32, K)

    def do_row(lp_hbm, sc_hbm, tv_hbm, ti_hbm, row):
        pltpu.sync_copy(lp_hbm.at[row], row_v)
        pltpu.sync_copy(sc_hbm.at[row], scv)
        s = scv[...]

        def group(g, carry):
            cv, ci, t = carry
            off = g * 256
            m = row_v[pl.ds(off, K)]
            for c in range(1, 16):
                m = jnp.maximum(m, row_v[pl.ds(off + c * K, K)])
            gmax = jnp.max(m + s)

            def hot(carry):
                def chunk(c, carry):
                    cv, ci, t = carry
                    coff = off + c * K
                    x = row_v[pl.ds(coff, K)] + s
                    cm = jnp.max(x)

                    def merge(_):
                        xs, xi = plsc.sort_key_val(
                            x, lane + coff, descending=True)
                        keep = cv >= xs
                        mv = jnp.where(keep, cv, xs)
                        mi = jnp.where(keep, ci, xi)
                        cv2, ci2 = plsc.sort_key_val(mv, mi, descending=False)
                        return cv2, ci2, jnp.min(cv2)

                    return lax.cond(cm > t, merge, lambda _: (cv, ci, t), 0)
                return lax.fori_loop(0, 16, chunk, carry)

            return lax.cond(gmax > t, hot, lambda c: c, (cv, ci, t))

        cv0 = jnp.full((K,), NEG, jnp.float32)
        ci0 = jnp.zeros((K,), jnp.int32)
        cv, ci, _ = lax.fori_loop(0, NGRP, group,
                                  (cv0, ci0, jnp.float32(NEG)))
        tmpv[...] = lax.rev(cv, (0,))
        tmpi[...] = lax.rev(ci, (0,))
        pltpu.sync_copy(tmpv, tv_hbm.at[row])
        pltpu.sync_copy(tmpi, ti_hbm.at[row])

    def rowloop(j, carry):
        do_row(lp0_hbm, sc0_hbm, tv0_hbm, ti0_hbm, base + j)
        do_row(lp1_hbm, sc1_hbm, tv1_hbm, ti1_hbm, base + j)
        return carry

    lax.fori_loop(0, RPW, rowloop, 0)


def _sc_topk(lp0, lp1, scb0, scb1):
    f32 = jnp.float32
    i32 = jnp.int32
    run = pl.kernel(
        _sc_body,
        out_type=(
            jax.ShapeDtypeStruct((NROW, K), f32),
            jax.ShapeDtypeStruct((NROW, K), i32),
            jax.ShapeDtypeStruct((NROW, K), f32),
            jax.ShapeDtypeStruct((NROW, K), i32),
        ),
        mesh=plsc.VectorSubcoreMesh(core_axis_name="c", subcore_axis_name="s"),
        compiler_params=pltpu.CompilerParams(needs_layout_passes=False),
        scratch_types=[
            pltpu.VMEM((V,), f32),
            pltpu.VMEM((K,), f32),
            pltpu.VMEM((K,), f32),
            pltpu.VMEM((K,), i32),
        ],
    )
    return run(lp0, lp1, scb0, scb1)


def _combine_body(tv0_ref, ti0_ref, tv1_ref, ti1_ref,
                  s0_ref, s1_ref, t0_ref, t1_ref, ib_ref):
    tv0 = tv0_ref[0]
    ti0 = ti0_ref[0]
    tv1 = tv1_ref[0]
    ti1 = ti1_ref[0]

    lane16 = jax.lax.broadcasted_iota(jnp.int32, (1, K), 1)
    oh16 = [lane16 == t for t in range(K)]
    ss = tv0[:, :, None] + tv1[:, None, :]
    fidx = (jax.lax.broadcasted_iota(jnp.int32, (BEAM, K, K), 0) * (K * K)
            + jax.lax.broadcasted_iota(jnp.int32, (BEAM, K, K), 1) * K
            + jax.lax.broadcasted_iota(jnp.int32, (BEAM, K, K), 2))
    beam_i = jax.lax.broadcasted_iota(jnp.int32, (BEAM, K), 0)
    col_i = jax.lax.broadcasted_iota(jnp.int32, (BEAM, K), 1)

    s0a = jnp.zeros((1, K), jnp.float32)
    s1a = jnp.zeros((1, K), jnp.float32)
    t0a = jnp.zeros((1, K), jnp.int32)
    t1a = jnp.zeros((1, K), jnp.int32)
    iba = jnp.zeros((1, K), jnp.int32)
    for t in range(K):
        m = jnp.max(ss)
        idx = jnp.min(jnp.where(ss == m, fidx, BIG))
        ss = jnp.where(fidx == idx, NEG, ss)
        ib = idx >> 8
        rem = idx & 255
        i0 = rem >> 4
        i1 = rem & 15
        sel0 = (beam_i == ib) & (col_i == i0)
        sel1 = (beam_i == ib) & (col_i == i1)
        v0 = jnp.sum(jnp.where(sel0, tv0, 0.0))
        n0 = jnp.sum(jnp.where(sel0, ti0, 0))
        v1 = jnp.sum(jnp.where(sel1, tv1, 0.0))
        n1 = jnp.sum(jnp.where(sel1, ti1, 0))
        oh = oh16[t]
        s0a = s0a + jnp.where(oh, v0, 0.0)
        s1a = s1a + jnp.where(oh, v1, 0.0)
        t0a = t0a + jnp.where(oh, n0, 0)
        t1a = t1a + jnp.where(oh, n1, 0)
        iba = iba + jnp.where(oh, ib, 0)

    s0_ref[0] = s0a
    s1_ref[0] = s1a
    t0_ref[0] = t0a
    t1_ref[0] = t1a
    ib_ref[0] = iba


def _tc_combine(tv0, ti0, tv1, ti1):
    out_shapes = tuple(
        jax.ShapeDtypeStruct((BSZ, 1, K), dt)
        for dt in (jnp.float32, jnp.float32, jnp.int32, jnp.int32, jnp.int32))
    spec = pl.BlockSpec((1, BEAM, K), lambda b: (b, 0, 0))
    out_spec = pl.BlockSpec((1, 1, K), lambda b: (b, 0, 0))
    return pl.pallas_call(
        _combine_body,
        grid=(BSZ,),
        in_specs=[spec] * 4,
        out_specs=(out_spec,) * 5,
        out_shape=out_shapes,
        compiler_params=pltpu.CompilerParams(
            dimension_semantics=("arbitrary",),
        ),
    )(tv0, ti0, tv1, ti1)


def kernel(step, lprobs_ch0, lprobs_ch1, scores_ch0, scores_ch1):
    sc0 = jax.lax.dynamic_index_in_dim(scores_ch0, step - 1, axis=2,
                                       keepdims=False)
    sc1 = jax.lax.dynamic_index_in_dim(scores_ch1, step - 1, axis=2,
                                       keepdims=False)
    lp0 = lprobs_ch0.reshape(NROW, V)
    lp1 = lprobs_ch1.reshape(NROW, V)
    scb0 = jnp.broadcast_to(sc0.reshape(NROW, 1), (NROW, K))
    scb1 = jnp.broadcast_to(sc1.reshape(NROW, 1), (NROW, K))

    tv0, ti0, tv1, ti1 = _sc_topk(lp0, lp1, scb0, scb1)

    s0, s1, t0, t1, ib = _tc_combine(
        tv0.reshape(BSZ, BEAM, K), ti0.reshape(BSZ, BEAM, K),
        tv1.reshape(BSZ, BEAM, K), ti1.reshape(BSZ, BEAM, K))
    return (s0[:, 0, :], s1[:, 0, :], t0[:, 0, :], t1[:, 0, :], ib[:, 0, :])
